# Initial kernel scaffold; baseline (speedup 1.0000x reference)
import jax
import jax.numpy as jnp
from jax.experimental import pallas as pl


def kernel(Rnl, Ylm, node_species, senders, receivers, W):
    raise NotImplementedError("write your pallas kernel here")


# speedup vs baseline: 17.3306x; 17.3306x over previous
import dataclasses
import functools

import jax
import jax.numpy as jnp
from jax import lax
from jax.experimental import pallas as pl
from jax.experimental.pallas import tpu as pltpu
from jax.experimental.pallas import tpu_sc as plsc

NUM_CORES = 2
NUM_SUBCORES = 16
NUM_WORKERS = NUM_CORES * NUM_SUBCORES
LANES = 16

SB = 80
CH = 2000
NSTREAM = CH // SB


def _sc_partial_sums(rnl, recv2d, zeros8, n_pad, n_edges):
    epw = n_edges // NUM_WORKERS
    nchunk = epw // CH
    rows_per_sub = n_pad // NUM_SUBCORES
    idx_rows_per_chunk = CH // SB

    mesh = plsc.VectorSubcoreMesh(core_axis_name="c", subcore_axis_name="s")
    out_t = jax.ShapeDtypeStruct((NUM_CORES, n_pad, 8), jnp.float32)
    cp = pltpu.CompilerParams()
    if "needs_layout_passes" in pltpu.CompilerParams.__dataclass_fields__:
        cp = dataclasses.replace(cp, needs_layout_passes=False)
    if "use_tc_tiling_on_sc" in pltpu.CompilerParams.__dataclass_fields__:
        cp = dataclasses.replace(cp, use_tc_tiling_on_sc=False)

    @functools.partial(
        pl.kernel,
        out_type=(out_t, out_t),
        mesh=mesh,
        compiler_params=cp,
        scratch_types=[
            pltpu.VMEM((CH, 8), jnp.float32),
            pltpu.VMEM((CH, 8), jnp.float32),
            pltpu.VMEM((idx_rows_per_chunk, SB), jnp.int32),
            pltpu.VMEM_SHARED((n_pad, 8), jnp.float32),
            pltpu.VMEM_SHARED((n_pad, 8), jnp.float32),
        ],
    )
    def seg_kernel(rnl_hbm, recv_hbm, zeros_hbm, out_raw, out_prod,
                   rnl_v, prod_v, idx_v, acc_raw, acc_prod):
        c = lax.axis_index("c")
        s = lax.axis_index("s")
        wid = c * NUM_SUBCORES + s

        stripe = pl.ds(s * rows_per_sub, rows_per_sub)
        pltpu.sync_copy(zeros_hbm.at[pl.ds(0, rows_per_sub)], acc_raw.at[stripe])
        pltpu.sync_copy(zeros_hbm.at[pl.ds(0, rows_per_sub)], acc_prod.at[stripe])
        pltpu.sync_copy(zeros_hbm.at[pl.ds(0, CH)], prod_v)
        plsc.subcore_barrier()

        iota = lax.iota(jnp.int32, LANES)
        col0 = jnp.zeros((LANES,), jnp.int32)
        col1 = jnp.full((LANES,), 1, jnp.int32)
        col2 = jnp.full((LANES,), 2, jnp.int32)

        e_base = wid * epw

        @pl.loop(0, nchunk)
        def _chunk(ci):
            e0 = e_base + ci * CH
            pltpu.sync_copy(rnl_hbm.at[pl.ds(e0, CH)], rnl_v)
            pltpu.sync_copy(recv_hbm.at[wid * nchunk + ci], idx_v)

            @pl.loop(0, CH // LANES)
            def _group(g):
                eidx = iota + g * LANES
                r0 = plsc.load_gather(rnl_v, [eidx, col0])
                r1 = plsc.load_gather(rnl_v, [eidx, col1])
                r2 = plsc.load_gather(rnl_v, [eidx, col2])
                plsc.store_scatter(prod_v, [eidx, col0], r0 * r1)
                plsc.store_scatter(prod_v, [eidx, col1], r0 * r2)
                plsc.store_scatter(prod_v, [eidx, col2], r1 * r2)

            @pl.loop(0, NSTREAM)
            def _stream(j):
                rows = pl.ds(j * SB, SB)
                pltpu.sync_copy(rnl_v.at[rows], acc_raw.at[idx_v.at[j]], add=True)
                pltpu.sync_copy(prod_v.at[rows], acc_prod.at[idx_v.at[j]], add=True)

        plsc.subcore_barrier()
        pltpu.sync_copy(acc_raw.at[stripe], out_raw.at[c].at[stripe])
        pltpu.sync_copy(acc_prod.at[stripe], out_prod.at[c].at[stripe])

    return seg_kernel(rnl, recv2d, zeros8)


def _tc_combine(raw_p, prod_p, n_pad):
    bn = 3128

    def body(raw_ref, prod_ref, o_ref):
        raw = raw_ref[0] + raw_ref[1]
        prod = prod_ref[0] + prod_ref[1]
        p01 = prod[:, 0:1]
        p02 = prod[:, 1:2]
        p12 = prod[:, 2:3]
        o_ref[...] = jnp.concatenate([raw, p01, p02, p01, p12, p02, p12], axis=1)

    return pl.pallas_call(
        body,
        grid=(n_pad // bn,),
        in_specs=[
            pl.BlockSpec((NUM_CORES, bn, 8), lambda i: (0, i, 0)),
            pl.BlockSpec((NUM_CORES, bn, 8), lambda i: (0, i, 0)),
        ],
        out_specs=pl.BlockSpec((bn, 14), lambda i: (i, 0)),
        out_shape=jax.ShapeDtypeStruct((n_pad, 14), jnp.float32),
    )(raw_p, prod_p)


def kernel(Rnl, Ylm, node_species, senders, receivers, W):
    n_edges = Rnl.shape[0]
    n_nodes = node_species.shape[0]
    n_pad = -(-n_nodes // (8 * NUM_SUBCORES)) * (8 * NUM_SUBCORES)
    recv2d = receivers.astype(jnp.int32).reshape(n_edges // CH, CH // SB, SB)
    zeros8 = jnp.zeros((n_pad // NUM_SUBCORES, 8), jnp.float32)
    raw_p, prod_p = _sc_partial_sums(Rnl, recv2d, zeros8, n_pad, n_edges)
    return _tc_combine(raw_p, prod_p, n_pad)[:n_nodes]


# speedup vs baseline: 18.1585x; 1.0478x over previous
import dataclasses
import functools

import jax
import jax.numpy as jnp
from jax import lax
from jax.experimental import pallas as pl
from jax.experimental.pallas import tpu as pltpu
from jax.experimental.pallas import tpu_sc as plsc

NUM_CORES = 2
NUM_SUBCORES = 16
NUM_WORKERS = NUM_CORES * NUM_SUBCORES
LANES = 16

SB = 80
CH = 2000
NSTREAM = CH // SB


def _sc_partial_sums(rnl, recv, zeros8, n_pad, n_edges):
    epw = n_edges // NUM_WORKERS
    nchunk = epw // CH
    rows_per_sub = n_pad // NUM_SUBCORES

    mesh = plsc.VectorSubcoreMesh(core_axis_name="c", subcore_axis_name="s")
    out_t = jax.ShapeDtypeStruct((NUM_CORES, n_pad, 8), jnp.float32)
    cp = pltpu.CompilerParams()
    if "needs_layout_passes" in pltpu.CompilerParams.__dataclass_fields__:
        cp = dataclasses.replace(cp, needs_layout_passes=False)
    if "use_tc_tiling_on_sc" in pltpu.CompilerParams.__dataclass_fields__:
        cp = dataclasses.replace(cp, use_tc_tiling_on_sc=False)

    @functools.partial(
        pl.kernel,
        out_type=(out_t, out_t),
        mesh=mesh,
        compiler_params=cp,
        scratch_types=[
            pltpu.VMEM((CH, 8), jnp.float32),
            pltpu.VMEM((CH, 8), jnp.float32),
            pltpu.VMEM((CH,), jnp.int32),
            pltpu.VMEM_SHARED((n_pad, 8), jnp.float32),
            pltpu.VMEM_SHARED((n_pad, 8), jnp.float32),
        ],
    )
    def seg_kernel(rnl_hbm, recv_hbm, zeros_hbm, out_raw, out_prod,
                   rnl_v, prod_v, idx_v, acc_raw, acc_prod):
        c = lax.axis_index("c")
        s = lax.axis_index("s")
        wid = c * NUM_SUBCORES + s

        stripe = pl.ds(s * rows_per_sub, rows_per_sub)
        pltpu.sync_copy(zeros_hbm.at[pl.ds(0, rows_per_sub)], acc_raw.at[stripe])
        pltpu.sync_copy(zeros_hbm.at[pl.ds(0, rows_per_sub)], acc_prod.at[stripe])
        pltpu.sync_copy(zeros_hbm.at[pl.ds(0, CH)], prod_v)
        plsc.subcore_barrier()

        iota = lax.iota(jnp.int32, LANES)
        cols = [jnp.full((LANES,), k, jnp.int32) for k in range(6)]

        e_base = wid * epw

        @pl.loop(0, nchunk)
        def _chunk(ci):
            e0 = e_base + ci * CH
            pltpu.sync_copy(rnl_hbm.at[pl.ds(e0, CH)], rnl_v)
            pltpu.sync_copy(recv_hbm.at[pl.ds(e0, CH)], idx_v)

            @pl.loop(0, CH // LANES)
            def _group(g):
                eidx = iota + g * LANES
                r0 = plsc.load_gather(rnl_v, [eidx, cols[0]])
                r1 = plsc.load_gather(rnl_v, [eidx, cols[1]])
                r2 = plsc.load_gather(rnl_v, [eidx, cols[2]])
                p01 = r0 * r1
                p02 = r0 * r2
                p12 = r1 * r2
                plsc.store_scatter(prod_v, [eidx, cols[0]], p01)
                plsc.store_scatter(prod_v, [eidx, cols[1]], p02)
                plsc.store_scatter(prod_v, [eidx, cols[2]], p01)
                plsc.store_scatter(prod_v, [eidx, cols[3]], p12)
                plsc.store_scatter(prod_v, [eidx, cols[4]], p02)
                plsc.store_scatter(prod_v, [eidx, cols[5]], p12)

            @pl.loop(0, NSTREAM)
            def _stream(j):
                rows = pl.ds(j * SB, SB)
                idxs = idx_v.at[pl.ds(j * SB, SB)]
                pltpu.sync_copy(rnl_v.at[rows], acc_raw.at[idxs], add=True)
                pltpu.sync_copy(prod_v.at[rows], acc_prod.at[idxs], add=True)

        plsc.subcore_barrier()
        pltpu.sync_copy(acc_raw.at[stripe], out_raw.at[c].at[stripe])
        pltpu.sync_copy(acc_prod.at[stripe], out_prod.at[c].at[stripe])

    return seg_kernel(rnl, recv, zeros8)


def _tc_combine(raw_p, prod_p, n_pad):
    rows = n_pad // 16

    def body(raw_ref, prod_ref, raw_o, prod_o):
        raw_o[...] = raw_ref[0] + raw_ref[1]
        prod_o[...] = prod_ref[0] + prod_ref[1]

    out_t = jax.ShapeDtypeStruct((rows, 128), jnp.float32)
    return pl.pallas_call(
        body,
        grid=(1,),
        in_specs=[
            pl.BlockSpec((NUM_CORES, rows, 128), lambda i: (0, 0, 0)),
            pl.BlockSpec((NUM_CORES, rows, 128), lambda i: (0, 0, 0)),
        ],
        out_specs=[
            pl.BlockSpec((rows, 128), lambda i: (0, 0)),
            pl.BlockSpec((rows, 128), lambda i: (0, 0)),
        ],
        out_shape=(out_t, out_t),
    )(raw_p.reshape(NUM_CORES, rows, 128), prod_p.reshape(NUM_CORES, rows, 128))


def kernel(Rnl, Ylm, node_species, senders, receivers, W):
    n_edges = Rnl.shape[0]
    n_nodes = node_species.shape[0]
    n_pad = -(-n_nodes // (8 * NUM_SUBCORES)) * (8 * NUM_SUBCORES)
    recv = receivers.astype(jnp.int32)
    zeros8 = jnp.zeros((n_pad // NUM_SUBCORES, 8), jnp.float32)
    raw_p, prod_p = _sc_partial_sums(Rnl, recv, zeros8, n_pad, n_edges)
    raw, prod = _tc_combine(raw_p, prod_p, n_pad)
    raw = raw.reshape(n_pad, 8)[:n_nodes]
    prod = prod.reshape(n_pad, 8)[:n_nodes, :6]
    return jnp.concatenate([raw, prod], axis=1)
